# baseline (speedup 1.0000x reference)
import functools
import math

import jax
import jax.numpy as jnp
import numpy as np
from jax import lax
from jax.experimental import pallas as pl
from jax.experimental.pallas import tpu as pltpu
from jax.experimental.pallas import tpu_sc as plsc


def _make_pe_np(max_length: int, d: int) -> np.ndarray:
    pe = np.zeros((max_length, d), dtype=np.float32)
    position = np.arange(0.0, max_length, dtype=np.float32)[:, None]
    div_term = np.exp(np.arange(0.0, d, 2, dtype=np.float32) * -(math.log(10000.0) / d))
    pe[:, 0::2] = np.sin(position * div_term)
    pe[:, 1::2] = np.cos(position * div_term)
    return pe


@functools.lru_cache(maxsize=None)
def _sc_gather_fn(n: int, v: int, d: int, chunk: int):
    info = plsc.get_sparse_core_info()
    nc, ns = info.num_cores, info.num_subcores
    nw = nc * ns
    assert n % (nw * 2 * chunk) == 0 and chunk % 8 == 0
    b_per_w = n // nw
    n_pairs = b_per_w // (2 * chunk)

    mesh = plsc.VectorSubcoreMesh(core_axis_name="c", subcore_axis_name="s")

    @functools.partial(
        pl.kernel,
        mesh=mesh,
        out_type=jax.ShapeDtypeStruct((n, d), jnp.float32),
        scratch_types=[
            pltpu.VMEM((b_per_w,), jnp.int32),
            pltpu.VMEM((chunk, d), jnp.float32),
            pltpu.VMEM((chunk, d), jnp.float32),
            pltpu.SemaphoreType.DMA,
            pltpu.SemaphoreType.DMA,
            pltpu.SemaphoreType.DMA,
        ],
    )
    def sc_gather(table_hbm, idx_hbm, out_hbm, idx_v, rows0, rows1, gsem, w0, w1):
        wid = lax.axis_index("s") * nc + lax.axis_index("c")
        base = wid * b_per_w
        pltpu.sync_copy(idx_hbm.at[pl.ds(base, b_per_w)], idx_v)
        rows = (rows0, rows1)
        wsem = (w0, w1)

        def pair(j, carry):
            for s in range(2):
                i = 2 * j + s
                off = pl.multiple_of(base + i * chunk, 8)
                @pl.when(j > 0)
                def _():
                    prev = pl.multiple_of(base + (i - 2) * chunk, 8)
                    pltpu.make_async_copy(
                        rows[s], out_hbm.at[pl.ds(prev, chunk)], wsem[s]
                    ).wait()

                pltpu.async_copy(
                    table_hbm.at[idx_v.at[pl.ds(i * chunk, chunk)]], rows[s], gsem
                ).wait()
                pltpu.async_copy(rows[s], out_hbm.at[pl.ds(off, chunk)], wsem[s])
            return carry

        lax.fori_loop(0, n_pairs, pair, 0)
        for s in range(2):
            i = 2 * (n_pairs - 1) + s
            off = pl.multiple_of(base + i * chunk, 8)
            pltpu.make_async_copy(
                rows[s], out_hbm.at[pl.ds(off, chunk)], wsem[s]
            ).wait()

    return sc_gather


@functools.lru_cache(maxsize=None)
def _tc_dense_fn(b: int, l: int, d: int, bblk: int):
    assert b % bblk == 0
    scale = math.sqrt(float(d))

    def body(emb_ref, pe_ref, wt_ref, bias_ref, out_ref):
        x = emb_ref[...] * scale + pe_ref[...][None]
        xf = x.reshape(bblk * l, d)
        r = jnp.dot(xf, wt_ref[...], preferred_element_type=jnp.float32)
        r = r + bias_ref[...] + 1.0
        out_ref[...] = (xf * r).reshape(bblk, l, d)

    return pl.pallas_call(
        body,
        grid=(b // bblk,),
        in_specs=[
            pl.BlockSpec((bblk, l, d), lambda i: (i, 0, 0)),
            pl.BlockSpec((l, d), lambda i: (0, 0)),
            pl.BlockSpec((d, d), lambda i: (0, 0)),
            pl.BlockSpec((1, d), lambda i: (0, 0)),
        ],
        out_specs=pl.BlockSpec((bblk, l, d), lambda i: (i, 0, 0)),
        out_shape=jax.ShapeDtypeStruct((b, l, d), jnp.float32),
    )


def kernel(src, tok_embedding, linear_w, linear_b):
    b, l = src.shape
    v, d = tok_embedding.shape
    pe = jnp.asarray(_make_pe_np(512, d)[:l])
    wt = linear_w.T
    bias = linear_b.reshape(1, d)
    n_slices = 4
    bs = b // n_slices
    gather = _sc_gather_fn(bs * l, v, d, chunk=400)
    dense = _tc_dense_fn(bs, l, d, bblk=16)
    zs = []
    for k in range(n_slices):
        idx_k = src[k * bs:(k + 1) * bs].reshape(-1)
        emb_k = gather(tok_embedding, idx_k)
        zs.append(dense(emb_k.reshape(bs, l, d), pe, wt, bias))
    return jnp.concatenate(zs, axis=0)


# speedup vs baseline: 1.0001x; 1.0001x over previous
import functools
import math

import jax
import jax.numpy as jnp
import numpy as np
from jax import lax
from jax.experimental import pallas as pl
from jax.experimental.pallas import tpu as pltpu
from jax.experimental.pallas import tpu_sc as plsc


def _make_pe_np(max_length: int, d: int) -> np.ndarray:
    pe = np.zeros((max_length, d), dtype=np.float32)
    position = np.arange(0.0, max_length, dtype=np.float32)[:, None]
    div_term = np.exp(np.arange(0.0, d, 2, dtype=np.float32) * -(math.log(10000.0) / d))
    pe[:, 0::2] = np.sin(position * div_term)
    pe[:, 1::2] = np.cos(position * div_term)
    return pe


@functools.lru_cache(maxsize=None)
def _sc_gather_fn(n: int, v: int, d: int, chunk: int):
    info = plsc.get_sparse_core_info()
    nc, ns = info.num_cores, info.num_subcores
    nw = nc * ns
    assert n % (nw * 2 * chunk) == 0 and chunk % 8 == 0
    b_per_w = n // nw
    n_pairs = b_per_w // (2 * chunk)

    mesh = plsc.VectorSubcoreMesh(core_axis_name="c", subcore_axis_name="s")

    @functools.partial(
        pl.kernel,
        mesh=mesh,
        out_type=jax.ShapeDtypeStruct((n, d), jnp.float32),
        scratch_types=[
            pltpu.VMEM((b_per_w,), jnp.int32),
            pltpu.VMEM((chunk, d), jnp.float32),
            pltpu.VMEM((chunk, d), jnp.float32),
            pltpu.SemaphoreType.DMA,
            pltpu.SemaphoreType.DMA,
            pltpu.SemaphoreType.DMA,
        ],
    )
    def sc_gather(table_hbm, idx_hbm, out_hbm, idx_v, rows0, rows1, gsem, w0, w1):
        wid = lax.axis_index("s") * nc + lax.axis_index("c")
        base = wid * b_per_w
        pltpu.sync_copy(idx_hbm.at[pl.ds(base, b_per_w)], idx_v)
        rows = (rows0, rows1)
        wsem = (w0, w1)

        def pair(j, carry):
            for s in range(2):
                i = 2 * j + s
                off = pl.multiple_of(base + i * chunk, 8)
                @pl.when(j > 0)
                def _():
                    prev = pl.multiple_of(base + (i - 2) * chunk, 8)
                    pltpu.make_async_copy(
                        rows[s], out_hbm.at[pl.ds(prev, chunk)], wsem[s]
                    ).wait()

                pltpu.async_copy(
                    table_hbm.at[idx_v.at[pl.ds(i * chunk, chunk)]], rows[s], gsem
                ).wait()
                pltpu.async_copy(rows[s], out_hbm.at[pl.ds(off, chunk)], wsem[s])
            return carry

        lax.fori_loop(0, n_pairs, pair, 0)
        for s in range(2):
            i = 2 * (n_pairs - 1) + s
            off = pl.multiple_of(base + i * chunk, 8)
            pltpu.make_async_copy(
                rows[s], out_hbm.at[pl.ds(off, chunk)], wsem[s]
            ).wait()

    return sc_gather


@functools.lru_cache(maxsize=None)
def _tc_dense_fn(b: int, l: int, d: int, bblk: int):
    assert b % bblk == 0
    scale = math.sqrt(float(d))

    def body(emb_ref, pe_ref, wt_ref, bias_ref, out_ref):
        x = emb_ref[...] * scale + pe_ref[...][None]
        xf = x.reshape(bblk * l, d)
        r = jnp.dot(xf, wt_ref[...], preferred_element_type=jnp.float32)
        r = r + bias_ref[...] + 1.0
        out_ref[...] = (xf * r).reshape(bblk, l, d)

    return pl.pallas_call(
        body,
        grid=(b // bblk,),
        in_specs=[
            pl.BlockSpec((bblk, l, d), lambda i: (i, 0, 0)),
            pl.BlockSpec((l, d), lambda i: (0, 0)),
            pl.BlockSpec((d, d), lambda i: (0, 0)),
            pl.BlockSpec((1, d), lambda i: (0, 0)),
        ],
        out_specs=pl.BlockSpec((bblk, l, d), lambda i: (i, 0, 0)),
        out_shape=jax.ShapeDtypeStruct((b, l, d), jnp.float32),
    )


def kernel(src, tok_embedding, linear_w, linear_b):
    b, l = src.shape
    v, d = tok_embedding.shape
    pe = jnp.asarray(_make_pe_np(512, d)[:l])
    wt = linear_w.T
    bias = linear_b.reshape(1, d)
    n_slices = 4
    bs = b // n_slices
    gather = _sc_gather_fn(bs * l, v, d, chunk=400)
    dense = _tc_dense_fn(bs, l, d, bblk=16)
    embs = [
        gather(tok_embedding, src[k * bs:(k + 1) * bs].reshape(-1))
        for k in range(n_slices)
    ]
    zs = [dense(e.reshape(bs, l, d), pe, wt, bias) for e in embs]
    return jnp.concatenate(zs, axis=0)


# speedup vs baseline: 1.4424x; 1.4423x over previous
import functools
import math

import jax
import jax.numpy as jnp
import numpy as np
from jax import lax
from jax.experimental import pallas as pl
from jax.experimental.pallas import tpu as pltpu
from jax.experimental.pallas import tpu_sc as plsc


def _make_pe_np(max_length: int, d: int) -> np.ndarray:
    pe = np.zeros((max_length, d), dtype=np.float32)
    position = np.arange(0.0, max_length, dtype=np.float32)[:, None]
    div_term = np.exp(np.arange(0.0, d, 2, dtype=np.float32) * -(math.log(10000.0) / d))
    pe[:, 0::2] = np.sin(position * div_term)
    pe[:, 1::2] = np.cos(position * div_term)
    return pe


@functools.lru_cache(maxsize=None)
def _sc_gather_fn(n: int, v: int, d: int, chunk: int):
    info = plsc.get_sparse_core_info()
    nc, ns = info.num_cores, info.num_subcores
    nw = nc * ns
    assert n % (nw * 2 * chunk) == 0 and chunk % 8 == 0
    b_per_w = n // nw
    n_pairs = b_per_w // (2 * chunk)

    mesh = plsc.VectorSubcoreMesh(core_axis_name="c", subcore_axis_name="s")

    @functools.partial(
        pl.kernel,
        mesh=mesh,
        out_type=jax.ShapeDtypeStruct((n, d), jnp.float32),
        scratch_types=[
            pltpu.VMEM((b_per_w,), jnp.int32),
            pltpu.VMEM((chunk, d), jnp.float32),
            pltpu.VMEM((chunk, d), jnp.float32),
            pltpu.SemaphoreType.DMA,
            pltpu.SemaphoreType.DMA,
            pltpu.SemaphoreType.DMA,
        ],
    )
    def sc_gather(table_hbm, idx_hbm, out_hbm, idx_v, rows0, rows1, gsem, w0, w1):
        wid = lax.axis_index("s") * nc + lax.axis_index("c")
        base = wid * b_per_w
        pltpu.sync_copy(idx_hbm.at[pl.ds(base, b_per_w)], idx_v)
        rows = (rows0, rows1)
        wsem = (w0, w1)

        def pair(j, carry):
            for s in range(2):
                i = 2 * j + s
                off = pl.multiple_of(base + i * chunk, 8)
                @pl.when(j > 0)
                def _():
                    prev = pl.multiple_of(base + (i - 2) * chunk, 8)
                    pltpu.make_async_copy(
                        rows[s], out_hbm.at[pl.ds(prev, chunk)], wsem[s]
                    ).wait()

                pltpu.async_copy(
                    table_hbm.at[idx_v.at[pl.ds(i * chunk, chunk)]], rows[s], gsem
                ).wait()
                pltpu.async_copy(rows[s], out_hbm.at[pl.ds(off, chunk)], wsem[s])
            return carry

        lax.fori_loop(0, n_pairs, pair, 0)
        for s in range(2):
            i = 2 * (n_pairs - 1) + s
            off = pl.multiple_of(base + i * chunk, 8)
            pltpu.make_async_copy(
                rows[s], out_hbm.at[pl.ds(off, chunk)], wsem[s]
            ).wait()

    return sc_gather


@functools.lru_cache(maxsize=None)
def _tc_dense_fn(b: int, l: int, d: int, bblk: int):
    assert b % bblk == 0
    scale = math.sqrt(float(d))

    def body(emb_ref, pe_ref, wt_ref, bias_ref, out_ref):
        x = emb_ref[...] * scale + pe_ref[...][None]
        xf = x.reshape(bblk * l, d)
        r = jnp.dot(xf, wt_ref[...], preferred_element_type=jnp.float32)
        r = r + bias_ref[...] + 1.0
        out_ref[...] = (xf * r).reshape(bblk, l, d)

    return pl.pallas_call(
        body,
        grid=(b // bblk,),
        in_specs=[
            pl.BlockSpec((bblk, l, d), lambda i: (i, 0, 0)),
            pl.BlockSpec((l, d), lambda i: (0, 0)),
            pl.BlockSpec((d, d), lambda i: (0, 0)),
            pl.BlockSpec((1, d), lambda i: (0, 0)),
        ],
        out_specs=pl.BlockSpec((bblk, l, d), lambda i: (i, 0, 0)),
        out_shape=jax.ShapeDtypeStruct((b, l, d), jnp.float32),
    )


def kernel(src, tok_embedding, linear_w, linear_b):
    b, l = src.shape
    v, d = tok_embedding.shape
    pe = jnp.asarray(_make_pe_np(512, d)[:l])
    wt = linear_w.T
    bias = linear_b.reshape(1, d)
    emb = _sc_gather_fn(b * l, v, d, chunk=400)(tok_embedding, src.reshape(-1))
    return _tc_dense_fn(b, l, d, bblk=32)(emb.reshape(b, l, d), pe, wt, bias)


# speedup vs baseline: 1.4713x; 1.0200x over previous
import functools
import math

import jax
import jax.numpy as jnp
import numpy as np
from jax import lax
from jax.experimental import pallas as pl
from jax.experimental.pallas import tpu as pltpu
from jax.experimental.pallas import tpu_sc as plsc


def _make_pe_np(max_length: int, d: int) -> np.ndarray:
    pe = np.zeros((max_length, d), dtype=np.float32)
    position = np.arange(0.0, max_length, dtype=np.float32)[:, None]
    div_term = np.exp(np.arange(0.0, d, 2, dtype=np.float32) * -(math.log(10000.0) / d))
    pe[:, 0::2] = np.sin(position * div_term)
    pe[:, 1::2] = np.cos(position * div_term)
    return pe


@functools.lru_cache(maxsize=None)
def _sc_gather_fn(n: int, v: int, d: int, chunk: int):
    info = plsc.get_sparse_core_info()
    nc, ns = info.num_cores, info.num_subcores
    nw = nc * ns
    assert n % (nw * 2 * chunk) == 0 and chunk % 8 == 0
    b_per_w = n // nw
    n_pairs = b_per_w // (2 * chunk)

    mesh = plsc.VectorSubcoreMesh(core_axis_name="c", subcore_axis_name="s")

    @functools.partial(
        pl.kernel,
        mesh=mesh,
        out_type=jax.ShapeDtypeStruct((n, d), jnp.float32),
        scratch_types=[
            pltpu.VMEM((b_per_w,), jnp.int32),
            pltpu.VMEM((chunk, d), jnp.float32),
            pltpu.VMEM((chunk, d), jnp.float32),
            pltpu.SemaphoreType.DMA,
            pltpu.SemaphoreType.DMA,
            pltpu.SemaphoreType.DMA,
        ],
    )
    def sc_gather(table_hbm, idx_hbm, out_hbm, idx_v, rows0, rows1, gsem, w0, w1):
        wid = lax.axis_index("s") * nc + lax.axis_index("c")
        base = wid * b_per_w
        pltpu.sync_copy(idx_hbm.at[pl.ds(base, b_per_w)], idx_v)
        rows = (rows0, rows1)
        wsem = (w0, w1)

        def pair(j, carry):
            for s in range(2):
                i = 2 * j + s
                off = pl.multiple_of(base + i * chunk, 8)
                @pl.when(j > 0)
                def _():
                    prev = pl.multiple_of(base + (i - 2) * chunk, 8)
                    pltpu.make_async_copy(
                        rows[s], out_hbm.at[pl.ds(prev, chunk)], wsem[s]
                    ).wait()

                pltpu.async_copy(
                    table_hbm.at[idx_v.at[pl.ds(i * chunk, chunk)]], rows[s], gsem
                ).wait()
                pltpu.async_copy(rows[s], out_hbm.at[pl.ds(off, chunk)], wsem[s])
            return carry

        lax.fori_loop(0, n_pairs, pair, 0)
        for s in range(2):
            i = 2 * (n_pairs - 1) + s
            off = pl.multiple_of(base + i * chunk, 8)
            pltpu.make_async_copy(
                rows[s], out_hbm.at[pl.ds(off, chunk)], wsem[s]
            ).wait()

    return sc_gather


@functools.lru_cache(maxsize=None)
def _tc_dense_fn(b: int, l: int, d: int, bblk: int):
    assert b % bblk == 0
    scale = math.sqrt(float(d))

    def body(emb_ref, pe_ref, wt_ref, bias_ref, out_ref):
        x = emb_ref[...] * scale + pe_ref[...][None]
        xf = x.reshape(bblk * l, d)
        r = jnp.dot(xf, wt_ref[...], preferred_element_type=jnp.float32)
        r = r + bias_ref[...] + 1.0
        out_ref[...] = (xf * r).reshape(bblk, l, d)

    return pl.pallas_call(
        body,
        grid=(b // bblk,),
        in_specs=[
            pl.BlockSpec((bblk, l, d), lambda i: (i, 0, 0)),
            pl.BlockSpec((l, d), lambda i: (0, 0)),
            pl.BlockSpec((d, d), lambda i: (0, 0)),
            pl.BlockSpec((1, d), lambda i: (0, 0)),
        ],
        out_specs=pl.BlockSpec((bblk, l, d), lambda i: (i, 0, 0)),
        out_shape=jax.ShapeDtypeStruct((b, l, d), jnp.float32),
    )


def kernel(src, tok_embedding, linear_w, linear_b):
    b, l = src.shape
    v, d = tok_embedding.shape
    pe = jnp.asarray(_make_pe_np(512, d)[:l])
    wt = linear_w.T
    bias = linear_b.reshape(1, d)
    emb = _sc_gather_fn(b * l, v, d, chunk=400)(tok_embedding, src.reshape(-1))
    return _tc_dense_fn(b, l, d, bblk=64)(emb.reshape(b, l, d), pe, wt, bias)


# speedup vs baseline: 1.4793x; 1.0055x over previous
import functools
import math

import jax
import jax.numpy as jnp
import numpy as np
from jax import lax
from jax.experimental import pallas as pl
from jax.experimental.pallas import tpu as pltpu
from jax.experimental.pallas import tpu_sc as plsc


def _make_pe_np(max_length: int, d: int) -> np.ndarray:
    pe = np.zeros((max_length, d), dtype=np.float32)
    position = np.arange(0.0, max_length, dtype=np.float32)[:, None]
    div_term = np.exp(np.arange(0.0, d, 2, dtype=np.float32) * -(math.log(10000.0) / d))
    pe[:, 0::2] = np.sin(position * div_term)
    pe[:, 1::2] = np.cos(position * div_term)
    return pe


@functools.lru_cache(maxsize=None)
def _sc_gather_fn(n: int, v: int, d: int, chunk: int):
    info = plsc.get_sparse_core_info()
    nc, ns = info.num_cores, info.num_subcores
    nw = nc * ns
    assert n % (nw * 2 * chunk) == 0 and chunk % 8 == 0
    b_per_w = n // nw
    n_pairs = b_per_w // (2 * chunk)

    mesh = plsc.VectorSubcoreMesh(core_axis_name="c", subcore_axis_name="s")

    @functools.partial(
        pl.kernel,
        mesh=mesh,
        out_type=jax.ShapeDtypeStruct((n, d), jnp.float32),
        scratch_types=[
            pltpu.VMEM((b_per_w,), jnp.int32),
            pltpu.VMEM((chunk, d), jnp.float32),
            pltpu.VMEM((chunk, d), jnp.float32),
            pltpu.SemaphoreType.DMA,
            pltpu.SemaphoreType.DMA,
            pltpu.SemaphoreType.DMA,
        ],
    )
    def sc_gather(table_hbm, idx_hbm, out_hbm, idx_v, rows0, rows1, gsem, w0, w1):
        wid = lax.axis_index("s") * nc + lax.axis_index("c")
        base = wid * b_per_w
        pltpu.sync_copy(idx_hbm.at[pl.ds(base, b_per_w)], idx_v)
        rows = (rows0, rows1)
        wsem = (w0, w1)

        def pair(j, carry):
            for s in range(2):
                i = 2 * j + s
                off = pl.multiple_of(base + i * chunk, 8)
                @pl.when(j > 0)
                def _():
                    prev = pl.multiple_of(base + (i - 2) * chunk, 8)
                    pltpu.make_async_copy(
                        rows[s], out_hbm.at[pl.ds(prev, chunk)], wsem[s]
                    ).wait()

                pltpu.async_copy(
                    table_hbm.at[idx_v.at[pl.ds(i * chunk, chunk)]], rows[s], gsem
                ).wait()
                pltpu.async_copy(rows[s], out_hbm.at[pl.ds(off, chunk)], wsem[s])
            return carry

        lax.fori_loop(0, n_pairs, pair, 0)
        for s in range(2):
            i = 2 * (n_pairs - 1) + s
            off = pl.multiple_of(base + i * chunk, 8)
            pltpu.make_async_copy(
                rows[s], out_hbm.at[pl.ds(off, chunk)], wsem[s]
            ).wait()

    return sc_gather


@functools.lru_cache(maxsize=None)
def _tc_dense_fn(b: int, l: int, d: int, bblk: int):
    assert b % bblk == 0
    scale = math.sqrt(float(d))

    def body(emb_ref, pe_ref, wt_ref, bias_ref, out_ref):
        x = emb_ref[...] * scale + pe_ref[...][None]
        xf = x.reshape(bblk * l, d)
        r = jnp.dot(xf, wt_ref[...], preferred_element_type=jnp.float32)
        r = r + bias_ref[...] + 1.0
        out_ref[...] = (xf * r).reshape(bblk, l, d)

    return pl.pallas_call(
        body,
        grid=(b // bblk,),
        in_specs=[
            pl.BlockSpec((bblk, l, d), lambda i: (i, 0, 0)),
            pl.BlockSpec((l, d), lambda i: (0, 0)),
            pl.BlockSpec((d, d), lambda i: (0, 0)),
            pl.BlockSpec((1, d), lambda i: (0, 0)),
        ],
        out_specs=pl.BlockSpec((bblk, l, d), lambda i: (i, 0, 0)),
        out_shape=jax.ShapeDtypeStruct((b, l, d), jnp.float32),
    )


def kernel(src, tok_embedding, linear_w, linear_b):
    b, l = src.shape
    v, d = tok_embedding.shape
    pe = jnp.asarray(_make_pe_np(512, d)[:l])
    wt = linear_w.T
    bias = linear_b.reshape(1, d)
    emb = _sc_gather_fn(b * l, v, d, chunk=400)(tok_embedding, src.reshape(-1))
    return _tc_dense_fn(b, l, d, bblk=128)(emb.reshape(b, l, d), pe, wt, bias)


# speedup vs baseline: 1.4893x; 1.0067x over previous
import functools
import math

import jax
import jax.numpy as jnp
import numpy as np
from jax import lax
from jax.experimental import pallas as pl
from jax.experimental.pallas import tpu as pltpu
from jax.experimental.pallas import tpu_sc as plsc


def _make_pe_np(max_length: int, d: int) -> np.ndarray:
    pe = np.zeros((max_length, d), dtype=np.float32)
    position = np.arange(0.0, max_length, dtype=np.float32)[:, None]
    div_term = np.exp(np.arange(0.0, d, 2, dtype=np.float32) * -(math.log(10000.0) / d))
    pe[:, 0::2] = np.sin(position * div_term)
    pe[:, 1::2] = np.cos(position * div_term)
    return pe


@functools.lru_cache(maxsize=None)
def _sc_gather_fn(n: int, v: int, d: int, chunk: int):
    info = plsc.get_sparse_core_info()
    nc, ns = info.num_cores, info.num_subcores
    nw = nc * ns
    assert n % (nw * 2 * chunk) == 0 and chunk % 8 == 0
    b_per_w = n // nw
    n_pairs = b_per_w // (2 * chunk)

    mesh = plsc.VectorSubcoreMesh(core_axis_name="c", subcore_axis_name="s")

    @functools.partial(
        pl.kernel,
        mesh=mesh,
        out_type=jax.ShapeDtypeStruct((n, d), jnp.float32),
        scratch_types=[
            pltpu.VMEM((b_per_w,), jnp.int32),
            pltpu.VMEM((chunk, d), jnp.float32),
            pltpu.VMEM((chunk, d), jnp.float32),
            pltpu.SemaphoreType.DMA,
            pltpu.SemaphoreType.DMA,
            pltpu.SemaphoreType.DMA,
        ],
    )
    def sc_gather(table_hbm, idx_hbm, out_hbm, idx_v, rows0, rows1, gsem, w0, w1):
        wid = lax.axis_index("s") * nc + lax.axis_index("c")
        base = wid * b_per_w
        pltpu.sync_copy(idx_hbm.at[pl.ds(base, b_per_w)], idx_v)
        rows = (rows0, rows1)
        wsem = (w0, w1)

        def pair(j, carry):
            for s in range(2):
                i = 2 * j + s
                off = pl.multiple_of(base + i * chunk, 8)
                @pl.when(j > 0)
                def _():
                    prev = pl.multiple_of(base + (i - 2) * chunk, 8)
                    pltpu.make_async_copy(
                        rows[s], out_hbm.at[pl.ds(prev, chunk)], wsem[s]
                    ).wait()

                pltpu.async_copy(
                    table_hbm.at[idx_v.at[pl.ds(i * chunk, chunk)]], rows[s], gsem
                ).wait()
                pltpu.async_copy(rows[s], out_hbm.at[pl.ds(off, chunk)], wsem[s])
            return carry

        lax.fori_loop(0, n_pairs, pair, 0)
        for s in range(2):
            i = 2 * (n_pairs - 1) + s
            off = pl.multiple_of(base + i * chunk, 8)
            pltpu.make_async_copy(
                rows[s], out_hbm.at[pl.ds(off, chunk)], wsem[s]
            ).wait()

    return sc_gather


@functools.lru_cache(maxsize=None)
def _tc_dense_fn(b: int, l: int, d: int, bblk: int):
    assert b % bblk == 0
    scale = math.sqrt(float(d))

    def body(emb_ref, pe_ref, wt_ref, bias_ref, out_ref):
        x = emb_ref[...] * scale + pe_ref[...][None]
        xf = x.reshape(bblk * l, d)
        r = jnp.dot(xf, wt_ref[...], preferred_element_type=jnp.float32)
        r = r + bias_ref[...] + 1.0
        out_ref[...] = (xf * r).reshape(bblk, l, d)

    return pl.pallas_call(
        body,
        grid=(b // bblk,),
        in_specs=[
            pl.BlockSpec((bblk, l, d), lambda i: (i, 0, 0)),
            pl.BlockSpec((l, d), lambda i: (0, 0)),
            pl.BlockSpec((d, d), lambda i: (0, 0)),
            pl.BlockSpec((1, d), lambda i: (0, 0)),
        ],
        out_specs=pl.BlockSpec((bblk, l, d), lambda i: (i, 0, 0)),
        out_shape=jax.ShapeDtypeStruct((b, l, d), jnp.float32),
    )


def kernel(src, tok_embedding, linear_w, linear_b):
    b, l = src.shape
    v, d = tok_embedding.shape
    pe = jnp.asarray(_make_pe_np(512, d)[:l])
    wt = linear_w.T
    bias = linear_b.reshape(1, d)
    emb = _sc_gather_fn(b * l, v, d, chunk=320)(tok_embedding, src.reshape(-1))
    return _tc_dense_fn(b, l, d, bblk=128)(emb.reshape(b, l, d), pe, wt, bias)
